# Initial kernel scaffold; baseline (speedup 1.0000x reference)
import jax
import jax.numpy as jnp
from jax.experimental import pallas as pl


def kernel(x, edge_index, W1, att_src1, att_dst1, b1, W2, att_src2, att_dst2, b2):
    raise NotImplementedError("write your pallas kernel here")


# speedup vs baseline: 58.0632x; 58.0632x over previous
import functools

import jax
import jax.numpy as jnp
from jax import lax
from jax.experimental import pallas as pl
from jax.experimental.pallas import tpu as pltpu
from jax.experimental.pallas import tpu_sc as plsc

N = 10000
NP = 10240
E = 320000
D_IN = 128
H1 = 8
C1 = 8
F1 = H1 * C1
F2 = 16

NC = 2
NS = 16
NW = NC * NS
EPW = E // NW
EB = 80
NB = EPW // EB
NPS = NP // NS

_f32 = jnp.float32
_i32 = jnp.int32


def _tc_a_body(x_ref, w1_ref, asbd_ref, adbd_ref, h1_ref, a1s_ref, a1d_ref):
    h = jnp.dot(x_ref[...], w1_ref[...], preferred_element_type=_f32)
    h1_ref[...] = h
    a1s_ref[...] = jnp.dot(h, asbd_ref[...], preferred_element_type=_f32)
    a1d_ref[...] = jnp.dot(h, adbd_ref[...], preferred_element_type=_f32)


def _tc_a(x, w1, as_bd, ad_bd):
    bn = 1280
    grid = NP // bn
    return pl.pallas_call(
        _tc_a_body,
        grid=(grid,),
        in_specs=[
            pl.BlockSpec((bn, D_IN), lambda i: (i, 0)),
            pl.BlockSpec((D_IN, F1), lambda i: (0, 0)),
            pl.BlockSpec((F1, H1), lambda i: (0, 0)),
            pl.BlockSpec((F1, H1), lambda i: (0, 0)),
        ],
        out_specs=[
            pl.BlockSpec((bn, F1), lambda i: (i, 0)),
            pl.BlockSpec((bn, H1), lambda i: (i, 0)),
            pl.BlockSpec((bn, H1), lambda i: (i, 0)),
        ],
        out_shape=[
            jax.ShapeDtypeStruct((NP, F1), _f32),
            jax.ShapeDtypeStruct((NP, H1), _f32),
            jax.ShapeDtypeStruct((NP, H1), _f32),
        ],
    )(x, w1, as_bd, ad_bd)


def _sc1_body(src_hbm, dst_hbm, a1s_hbm, a1d_hbm, h1_hbm, z64_hbm, z8_hbm,
              accp_hbm, denp_hbm,
              sp_acc, sp_den,
              ev_src, ev_dst, g_as, g_ad, wb, g_h1, msg,
              stage64, stage8, sem):
    c = lax.axis_index("c")
    s = lax.axis_index("s")
    wid = c * NS + s
    row0 = s * NPS

    pltpu.sync_copy(z64_hbm, stage64)
    pltpu.sync_copy(stage64, sp_acc.at[pl.ds(row0, NPS)])
    pltpu.sync_copy(z8_hbm, stage8)
    pltpu.sync_copy(stage8, sp_den.at[pl.ds(row0, NPS)])
    pltpu.sync_copy(src_hbm.at[wid], ev_src)
    pltpu.sync_copy(dst_hbm.at[wid], ev_dst)
    plsc.subcore_barrier()

    iota = jnp.arange(16, dtype=_i32)
    rows_lo = iota // 8
    cols8 = iota % 8

    @pl.loop(0, NB)
    def _blk(b):
        srcb = ev_src.at[b]
        dstb = ev_dst.at[b]
        d1 = pltpu.async_copy(a1s_hbm.at[srcb], g_as, sem)
        d2 = pltpu.async_copy(a1d_hbm.at[dstb], g_ad, sem)
        d3 = pltpu.async_copy(h1_hbm.at[srcb], g_h1, sem)
        d1.wait()
        d2.wait()
        d3.wait()
        for j in range(EB * H1 // 16):
            r = rows_lo + (2 * j)
            av = plsc.load_gather(g_as, [r, cols8])
            bv = plsc.load_gather(g_ad, [r, cols8])
            ev = av + bv
            wv = jnp.exp(jnp.maximum(ev, 0.2 * ev))
            plsc.store_scatter(wb, [r, cols8], wv)
        pltpu.sync_copy(wb, sp_den.at[dstb], add=True)

        @pl.loop(0, EB)
        def _edge(e):
            erow = jnp.zeros((16,), _i32) + e
            for q in range(4):
                wq = plsc.load_gather(wb, [erow, rows_lo + 2 * q])
                hq = g_h1[e, pl.ds(16 * q, 16)]
                msg[e, pl.ds(16 * q, 16)] = wq * hq
        pltpu.sync_copy(msg, sp_acc.at[dstb], add=True)

    plsc.subcore_barrier()
    pltpu.sync_copy(sp_acc.at[pl.ds(row0, NPS)], stage64)
    pltpu.sync_copy(stage64, accp_hbm.at[c, pl.ds(row0, NPS)])
    pltpu.sync_copy(sp_den.at[pl.ds(row0, NPS)], stage8)
    pltpu.sync_copy(stage8, denp_hbm.at[c, pl.ds(row0, NPS)])


def _sc_edge1(srcr, dstr, a1s, a1d, h1, z64, z8):
    mesh = plsc.VectorSubcoreMesh(core_axis_name="c", subcore_axis_name="s")
    fn = functools.partial(
        pl.kernel,
        out_type=[
            jax.ShapeDtypeStruct((NC, NP, F1), _f32),
            jax.ShapeDtypeStruct((NC, NP, H1), _f32),
        ],
        mesh=mesh,
        compiler_params=pltpu.CompilerParams(
            use_tc_tiling_on_sc=False, needs_layout_passes=False),
        scratch_types=[
            pltpu.VMEM_SHARED((NP, F1), _f32),
            pltpu.VMEM_SHARED((NP, H1), _f32),
            pltpu.VMEM((NB, EB), _i32),
            pltpu.VMEM((NB, EB), _i32),
            pltpu.VMEM((EB, H1), _f32),
            pltpu.VMEM((EB, H1), _f32),
            pltpu.VMEM((EB, H1), _f32),
            pltpu.VMEM((EB, F1), _f32),
            pltpu.VMEM((EB, F1), _f32),
            pltpu.VMEM((NPS, F1), _f32),
            pltpu.VMEM((NPS, H1), _f32),
            pltpu.SemaphoreType.DMA,
        ],
    )(_sc1_body)
    return fn(srcr, dstr, a1s, a1d, h1, z64, z8)


def _tc_b_body(accp_ref, denp_ref, b1_ref, w2_ref, as2_ref, ad2_ref, r8_ref,
               h2_ref, a2_ref):
    acc = accp_ref[0] + accp_ref[1]
    den = denp_ref[0] + denp_ref[1]
    dinv = 1.0 / (den + 1e-16)
    drep = jnp.dot(dinv, r8_ref[...], preferred_element_type=_f32)
    xact = acc * drep + b1_ref[...]
    act = jnp.where(xact > 0, xact, jnp.exp(xact) - 1.0)
    h2 = jnp.dot(act, w2_ref[...], preferred_element_type=_f32)
    h2_ref[...] = h2
    a2s = jnp.dot(h2, as2_ref[...], preferred_element_type=_f32)
    a2d = jnp.dot(h2, ad2_ref[...], preferred_element_type=_f32)
    a2_ref[...] = jnp.concatenate(
        [a2s, a2d, jnp.zeros_like(a2s), jnp.zeros_like(a2s),
         jnp.zeros_like(a2s), jnp.zeros_like(a2s), jnp.zeros_like(a2s),
         jnp.zeros_like(a2s)], axis=1)


def _tc_b(accp, denp, b1, w2, as2, ad2, r8):
    bn = 1280
    grid = NP // bn
    return pl.pallas_call(
        _tc_b_body,
        grid=(grid,),
        in_specs=[
            pl.BlockSpec((NC, bn, F1), lambda i: (0, i, 0)),
            pl.BlockSpec((NC, bn, H1), lambda i: (0, i, 0)),
            pl.BlockSpec((1, F1), lambda i: (0, 0)),
            pl.BlockSpec((F1, F2), lambda i: (0, 0)),
            pl.BlockSpec((F2, 1), lambda i: (0, 0)),
            pl.BlockSpec((F2, 1), lambda i: (0, 0)),
            pl.BlockSpec((H1, F1), lambda i: (0, 0)),
        ],
        out_specs=[
            pl.BlockSpec((bn, F2), lambda i: (i, 0)),
            pl.BlockSpec((bn, H1), lambda i: (i, 0)),
        ],
        out_shape=[
            jax.ShapeDtypeStruct((NP, F2), _f32),
            jax.ShapeDtypeStruct((NP, H1), _f32),
        ],
    )(accp, denp, b1, w2, as2, ad2, r8)


def _sc2_body(src_hbm, dst_hbm, h2_hbm, a2_hbm, z16_hbm, z8_hbm,
              accp_hbm, denp_hbm,
              sp_acc, sp_den,
              ev_src, ev_dst, g2s, g2d, wb2, g_h2, msg2,
              stage16, stage8, sem):
    c = lax.axis_index("c")
    s = lax.axis_index("s")
    wid = c * NS + s
    row0 = s * NPS

    pltpu.sync_copy(z16_hbm, stage16)
    pltpu.sync_copy(stage16, sp_acc.at[pl.ds(row0, NPS)])
    pltpu.sync_copy(z8_hbm, stage8)
    pltpu.sync_copy(stage8, sp_den.at[pl.ds(row0, NPS)])
    pltpu.sync_copy(src_hbm.at[wid], ev_src)
    pltpu.sync_copy(dst_hbm.at[wid], ev_dst)
    plsc.subcore_barrier()

    iota = jnp.arange(16, dtype=_i32)
    rows_lo = iota // 8
    cols8 = iota % 8
    zero16 = jnp.zeros((16,), _i32)

    @pl.loop(0, NB)
    def _blk(b):
        srcb = ev_src.at[b]
        dstb = ev_dst.at[b]
        d1 = pltpu.async_copy(a2_hbm.at[srcb], g2s, sem)
        d2 = pltpu.async_copy(a2_hbm.at[dstb], g2d, sem)
        d3 = pltpu.async_copy(h2_hbm.at[srcb], g_h2, sem)
        d1.wait()
        d2.wait()
        d3.wait()
        for j in range(EB * H1 // 16):
            r = rows_lo + (2 * j)
            sv = plsc.load_gather(g2s, [r, zero16])
            dv = plsc.load_gather(g2d, [r, zero16 + 1])
            ev = sv + dv
            wvv = jnp.exp(jnp.maximum(ev, 0.2 * ev))
            plsc.store_scatter(wb2, [r, cols8], wvv)
        pltpu.sync_copy(wb2, sp_den.at[dstb], add=True)

        @pl.loop(0, EB)
        def _edge(e):
            erow = zero16 + e
            ws = plsc.load_gather(wb2, [erow, zero16])
            msg2[e, :] = g_h2[e, :] * ws
        pltpu.sync_copy(msg2, sp_acc.at[dstb], add=True)

    plsc.subcore_barrier()
    pltpu.sync_copy(sp_acc.at[pl.ds(row0, NPS)], stage16)
    pltpu.sync_copy(stage16, accp_hbm.at[c, pl.ds(row0, NPS)])
    pltpu.sync_copy(sp_den.at[pl.ds(row0, NPS)], stage8)
    pltpu.sync_copy(stage8, denp_hbm.at[c, pl.ds(row0, NPS)])


def _sc_edge2(srcr, dstr, h2, a2, z16, z8):
    mesh = plsc.VectorSubcoreMesh(core_axis_name="c", subcore_axis_name="s")
    fn = functools.partial(
        pl.kernel,
        out_type=[
            jax.ShapeDtypeStruct((NC, NP, F2), _f32),
            jax.ShapeDtypeStruct((NC, NP, H1), _f32),
        ],
        mesh=mesh,
        compiler_params=pltpu.CompilerParams(
            use_tc_tiling_on_sc=False, needs_layout_passes=False),
        scratch_types=[
            pltpu.VMEM_SHARED((NP, F2), _f32),
            pltpu.VMEM_SHARED((NP, H1), _f32),
            pltpu.VMEM((NB, EB), _i32),
            pltpu.VMEM((NB, EB), _i32),
            pltpu.VMEM((EB, H1), _f32),
            pltpu.VMEM((EB, H1), _f32),
            pltpu.VMEM((EB, H1), _f32),
            pltpu.VMEM((EB, F2), _f32),
            pltpu.VMEM((EB, F2), _f32),
            pltpu.VMEM((NPS, F2), _f32),
            pltpu.VMEM((NPS, H1), _f32),
            pltpu.SemaphoreType.DMA,
        ],
    )(_sc2_body)
    return fn(srcr, dstr, h2, a2, z16, z8)


def _tc_c_body(accp_ref, denp_ref, b2_ref, out_ref):
    acc = accp_ref[0] + accp_ref[1]
    den = denp_ref[0][:, 0:1] + denp_ref[1][:, 0:1]
    logits = acc * (1.0 / (den + 1e-16)) + b2_ref[...]
    m = jnp.max(logits, axis=1, keepdims=True)
    sh = logits - m
    out_ref[...] = sh - jnp.log(jnp.sum(jnp.exp(sh), axis=1, keepdims=True))


def _tc_c(accp2, denp2, b2):
    return pl.pallas_call(
        _tc_c_body,
        grid=(1,),
        in_specs=[
            pl.BlockSpec((NC, NP, F2), lambda i: (0, 0, 0)),
            pl.BlockSpec((NC, NP, H1), lambda i: (0, 0, 0)),
            pl.BlockSpec((1, F2), lambda i: (0, 0)),
        ],
        out_specs=pl.BlockSpec((NP, F2), lambda i: (0, 0)),
        out_shape=jax.ShapeDtypeStruct((NP, F2), _f32),
    )(accp2, denp2, b2)


def kernel(x, edge_index, W1, att_src1, att_dst1, b1, W2, att_src2, att_dst2, b2):
    r8 = jnp.repeat(jnp.eye(H1, dtype=_f32), C1, axis=1)
    as_bd = r8.T * att_src1.reshape(-1)[:, None]
    ad_bd = r8.T * att_dst1.reshape(-1)[:, None]
    as2 = att_src2.reshape(F2, 1)
    ad2 = att_dst2.reshape(F2, 1)
    er = edge_index.reshape(2, NW, NB, EB)
    srcr, dstr = er[0], er[1]
    z64 = jnp.zeros((NPS, F1), _f32)
    z16 = jnp.zeros((NPS, F2), _f32)
    z8 = jnp.zeros((NPS, H1), _f32)

    xp = jnp.concatenate([x, jnp.zeros((NP - N, D_IN), _f32)], axis=0)
    h1, a1s, a1d = _tc_a(xp, W1, as_bd, ad_bd)
    accp, denp = _sc_edge1(srcr, dstr, a1s, a1d, h1, z64, z8)
    h2, a2 = _tc_b(accp, denp, b1.reshape(1, F1), W2, as2, ad2, r8)
    accp2, denp2 = _sc_edge2(srcr, dstr, h2, a2, z16, z8)
    out = _tc_c(accp2, denp2, b2.reshape(1, F2))
    return out[:N]


# speedup vs baseline: 80.8318x; 1.3921x over previous
import functools

import jax
import jax.numpy as jnp
from jax import lax
from jax.experimental import pallas as pl
from jax.experimental.pallas import tpu as pltpu
from jax.experimental.pallas import tpu_sc as plsc

N = 10000
NP = 10240
E = 320000
D_IN = 128
H1 = 8
C1 = 8
F1 = H1 * C1
G1 = F1 + H1
F2 = 16
G2 = F2 + H1

NC = 2
NS = 16
NW = NC * NS
EPW = E // NW
EB = 100
NB = EPW // EB
NPS = NP // NS
NST = NPS // 2

_f32 = jnp.float32
_i32 = jnp.int32


def _tc_a_body(x_ref, w1_ref, asbd_ref, adbd_ref, h1a_ref, a1d_ref):
    h = jnp.dot(x_ref[...], w1_ref[...], preferred_element_type=_f32)
    a1s = jnp.dot(h, asbd_ref[...], preferred_element_type=_f32)
    h1a_ref[...] = jnp.concatenate([h, a1s], axis=1)
    a1d_ref[...] = jnp.dot(h, adbd_ref[...], preferred_element_type=_f32)


def _tc_a(x, w1, as_bd, ad_bd):
    bn = 1280
    grid = NP // bn
    return pl.pallas_call(
        _tc_a_body,
        grid=(grid,),
        in_specs=[
            pl.BlockSpec((bn, D_IN), lambda i: (i, 0)),
            pl.BlockSpec((D_IN, F1), lambda i: (0, 0)),
            pl.BlockSpec((F1, H1), lambda i: (0, 0)),
            pl.BlockSpec((F1, H1), lambda i: (0, 0)),
        ],
        out_specs=[
            pl.BlockSpec((bn, G1), lambda i: (i, 0)),
            pl.BlockSpec((bn, H1), lambda i: (i, 0)),
        ],
        out_shape=[
            jax.ShapeDtypeStruct((NP, G1), _f32),
            jax.ShapeDtypeStruct((NP, H1), _f32),
        ],
    )(x, w1, as_bd, ad_bd)


def _sc_body(FW, GW,
             src_hbm, dst_hbm, tab_hbm, ad_hbm, z_hbm, accp_hbm,
             sp_acc, ev_src, ev_dst, g0, g1, gd0, gd1, m0, m1,
             stage, sg0, sg1):
    c = lax.axis_index("c")
    s = lax.axis_index("s")
    wid = c * NS + s
    row0 = s * NPS

    pltpu.sync_copy(z_hbm, stage)
    pltpu.sync_copy(stage, sp_acc.at[pl.ds(row0, NST)])
    pltpu.sync_copy(stage, sp_acc.at[pl.ds(row0 + NST, NST)])
    pltpu.sync_copy(src_hbm.at[wid], ev_src)
    pltpu.sync_copy(dst_hbm.at[wid], ev_dst)
    plsc.subcore_barrier()

    iota = jnp.arange(16, dtype=_i32)
    rows_lo = iota // 8
    cols8 = iota % 8

    gbuf = (g0, g1)
    gdbuf = (gd0, gd1)
    mbuf = (m0, m1)
    sems = (sg0, sg1)

    def issue(b, u):
        pltpu.async_copy(tab_hbm.at[ev_src.at[b]], gbuf[u], sems[u])
        pltpu.async_copy(ad_hbm.at[ev_dst.at[b]], gdbuf[u], sems[u])

    def drain(b, u):
        pltpu.make_async_copy(tab_hbm.at[ev_src.at[b]], gbuf[u], sems[u]).wait()
        pltpu.make_async_copy(ad_hbm.at[ev_dst.at[b]], gdbuf[u], sems[u]).wait()

    issue(0, 0)
    issue(1, 1)

    @pl.loop(0, NB // 2)
    def _blk(t):
        for u in range(2):
            b = 2 * t + u
            g, gd, m = gbuf[u], gdbuf[u], mbuf[u]
            drain(b, u)
            for j in range(EB * H1 // 16):
                r = rows_lo + (2 * j)
                av = plsc.load_gather(g, [r, cols8 + FW])
                bv = plsc.load_gather(gd, [r, cols8])
                ev = av + bv
                wv = jnp.exp(jnp.maximum(ev, 0.2 * ev))
                plsc.store_scatter(m, [r, cols8 + FW], wv)
            @pl.loop(0, EB)
            def _edge(e):
                erow = jnp.zeros((16,), _i32) + e
                for q in range(FW // 16):
                    cw = (rows_lo + 2 * q) if FW == F1 else cols8
                    wq = plsc.load_gather(m, [erow, cw + FW])
                    hq = g[e, pl.ds(16 * q, 16)]
                    m[e, pl.ds(16 * q, 16)] = wq * hq
            pltpu.sync_copy(m, sp_acc.at[ev_dst.at[b]], add=True)
            nxt = b + 2

            @pl.when(nxt < NB)
            def _():
                issue(nxt, u)

    plsc.subcore_barrier()
    pltpu.sync_copy(sp_acc.at[pl.ds(row0, NST)], stage)
    pltpu.sync_copy(stage, accp_hbm.at[c, pl.ds(row0, NST)])
    pltpu.sync_copy(sp_acc.at[pl.ds(row0 + NST, NST)], stage)
    pltpu.sync_copy(stage, accp_hbm.at[c, pl.ds(row0 + NST, NST)])


def _sc_edge(FW, GW, srcr, dstr, tab, ad, z):
    mesh = plsc.VectorSubcoreMesh(core_axis_name="c", subcore_axis_name="s")
    fn = functools.partial(
        pl.kernel,
        out_type=jax.ShapeDtypeStruct((NC, NP, GW), _f32),
        mesh=mesh,
        compiler_params=pltpu.CompilerParams(
            use_tc_tiling_on_sc=False, needs_layout_passes=False),
        scratch_types=[
            pltpu.VMEM_SHARED((NP, GW), _f32),
            pltpu.VMEM((NB, EB), _i32),
            pltpu.VMEM((NB, EB), _i32),
            pltpu.VMEM((EB, GW), _f32),
            pltpu.VMEM((EB, GW), _f32),
            pltpu.VMEM((EB, H1), _f32),
            pltpu.VMEM((EB, H1), _f32),
            pltpu.VMEM((EB, GW), _f32),
            pltpu.VMEM((EB, GW), _f32),
            pltpu.VMEM((NST, GW), _f32),
            pltpu.SemaphoreType.DMA,
            pltpu.SemaphoreType.DMA,
        ],
    )(functools.partial(_sc_body, FW, GW))
    return fn(srcr, dstr, tab, ad, z)


def _tc_b_body(accp_ref, b1_ref, w2_ref, as2_ref, ad2_ref, r8_ref,
               h2a_ref, a2d_ref):
    fused = accp_ref[0] + accp_ref[1]
    acc = fused[:, 0:F1]
    den = fused[:, F1:G1]
    dinv = 1.0 / (den + 1e-16)
    drep = jnp.dot(dinv, r8_ref[...], preferred_element_type=_f32)
    xact = acc * drep + b1_ref[...]
    act = jnp.where(xact > 0, xact, jnp.exp(xact) - 1.0)
    h2 = jnp.dot(act, w2_ref[...], preferred_element_type=_f32)
    a2s = jnp.dot(h2, as2_ref[...], preferred_element_type=_f32)
    a2d = jnp.dot(h2, ad2_ref[...], preferred_element_type=_f32)
    a2s8 = jnp.concatenate([a2s] * H1, axis=1)
    h2a_ref[...] = jnp.concatenate([h2, a2s8], axis=1)
    a2d_ref[...] = jnp.concatenate([a2d] * H1, axis=1)


def _tc_b(accp, b1, w2, as2, ad2, r8):
    bn = 1280
    grid = NP // bn
    return pl.pallas_call(
        _tc_b_body,
        grid=(grid,),
        in_specs=[
            pl.BlockSpec((NC, bn, G1), lambda i: (0, i, 0)),
            pl.BlockSpec((1, F1), lambda i: (0, 0)),
            pl.BlockSpec((F1, F2), lambda i: (0, 0)),
            pl.BlockSpec((F2, 1), lambda i: (0, 0)),
            pl.BlockSpec((F2, 1), lambda i: (0, 0)),
            pl.BlockSpec((H1, F1), lambda i: (0, 0)),
        ],
        out_specs=[
            pl.BlockSpec((bn, G2), lambda i: (i, 0)),
            pl.BlockSpec((bn, H1), lambda i: (i, 0)),
        ],
        out_shape=[
            jax.ShapeDtypeStruct((NP, G2), _f32),
            jax.ShapeDtypeStruct((NP, H1), _f32),
        ],
    )(accp, b1, w2, as2, ad2, r8)


def _tc_c_body(accp_ref, b2_ref, out_ref):
    fused = accp_ref[0] + accp_ref[1]
    acc = fused[:, 0:F2]
    den = fused[:, F2:F2 + 1]
    logits = acc * (1.0 / (den + 1e-16)) + b2_ref[...]
    m = jnp.max(logits, axis=1, keepdims=True)
    sh = logits - m
    out_ref[...] = sh - jnp.log(jnp.sum(jnp.exp(sh), axis=1, keepdims=True))


def _tc_c(accp2, b2):
    return pl.pallas_call(
        _tc_c_body,
        grid=(1,),
        in_specs=[
            pl.BlockSpec((NC, NP, G2), lambda i: (0, 0, 0)),
            pl.BlockSpec((1, F2), lambda i: (0, 0)),
        ],
        out_specs=pl.BlockSpec((NP, F2), lambda i: (0, 0)),
        out_shape=jax.ShapeDtypeStruct((NP, F2), _f32),
    )(accp2, b2)


def kernel(x, edge_index, W1, att_src1, att_dst1, b1, W2, att_src2, att_dst2, b2):
    r8 = jnp.repeat(jnp.eye(H1, dtype=_f32), C1, axis=1)
    as_bd = r8.T * att_src1.reshape(-1)[:, None]
    ad_bd = r8.T * att_dst1.reshape(-1)[:, None]
    as2 = att_src2.reshape(F2, 1)
    ad2 = att_dst2.reshape(F2, 1)
    er = edge_index.reshape(2, NW, NB, EB)
    srcr, dstr = er[0], er[1]
    z72 = jnp.zeros((NST, G1), _f32)
    z24 = jnp.zeros((NST, G2), _f32)

    xp = jnp.concatenate([x, jnp.zeros((NP - N, D_IN), _f32)], axis=0)
    h1a, a1d = _tc_a(xp, W1, as_bd, ad_bd)
    accp = _sc_edge(F1, G1, srcr, dstr, h1a, a1d, z72)
    h2a, a2d8 = _tc_b(accp, b1.reshape(1, F1), W2, as2, ad2, r8)
    accp2 = _sc_edge(F2, G2, srcr, dstr, h2a, a2d8, z24)
    out = _tc_c(accp2, b2.reshape(1, F2))
    return out[:N]
